# baseline (speedup 1.0000x reference)
import functools

import jax
import jax.numpy as jnp
from jax import lax
from jax.experimental import pallas as pl
from jax.experimental.pallas import tpu as pltpu
from jax.experimental.pallas import tpu_sc as plsc

_VOCAB = 1000000
_DIM = 64
_BATCH = 16384

_INFO = plsc.get_sparse_core_info()
_NC = _INFO.num_cores
_NS = _INFO.num_subcores
_L = _INFO.num_lanes
_NW = _NC * _NS
_BPW = _BATCH // _NW
_NCHUNK = _BPW // _L
_NBUF = 6


def _glove_kernel(word_ids_hbm, ctx_ids_hbm, wembT_hbm, cembT_hbm,
                  wbT_hbm, cbT_hbm, out_hbm,
                  widx_v, cidx_v, wslabs, cslabs, wbslabs, cbslabs,
                  out_v, sem):
    wid = lax.axis_index("s") * _NC + lax.axis_index("c")
    base = wid * _BPW

    pltpu.sync_copy(word_ids_hbm.at[pl.ds(base, _BPW)], widx_v)
    pltpu.sync_copy(ctx_ids_hbm.at[pl.ds(base, _BPW)], cidx_v)

    def fire(w, c, slot):
        bw = pl.multiple_of((w >> 7) * 128, 128)
        bc = pl.multiple_of((c >> 7) * 128, 128)
        pltpu.async_copy(wembT_hbm.at[:, pl.ds(bw, 128)], wslabs.at[slot], sem)
        pltpu.async_copy(cembT_hbm.at[:, pl.ds(bc, 128)], cslabs.at[slot], sem)
        pltpu.async_copy(wbT_hbm.at[:, pl.ds(bw, 128)], wbslabs.at[slot], sem)
        pltpu.async_copy(cbT_hbm.at[:, pl.ds(bc, 128)], cbslabs.at[slot], sem)

    def drain(slot):
        pltpu.make_async_copy(
            wembT_hbm.at[:, pl.ds(0, 128)], wslabs.at[slot], sem).wait()
        pltpu.make_async_copy(
            cembT_hbm.at[:, pl.ds(0, 128)], cslabs.at[slot], sem).wait()
        pltpu.make_async_copy(
            wbT_hbm.at[:, pl.ds(0, 128)], wbslabs.at[slot], sem).wait()
        pltpu.make_async_copy(
            cbT_hbm.at[:, pl.ds(0, 128)], cbslabs.at[slot], sem).wait()

    lane = lax.iota(jnp.int32, _L)
    zero16 = jnp.zeros((_L,), jnp.int32)

    wvec0 = widx_v[pl.ds(0, _L)]
    cvec0 = cidx_v[pl.ds(0, _L)]
    for b in range(_NBUF):
        fire(wvec0[b], cvec0[b], b)

    def chunk_body(cc, carry):
        o = cc * _L
        wvec = widx_v[pl.ds(o, _L)]
        cvec = cidx_v[pl.ds(o, _L)]
        last = cc == _NCHUNK - 1
        onxt = jnp.where(last, o, o + _L)
        wnxt = widx_v[pl.ds(onxt, _L)]
        cnxt = cidx_v[pl.ds(onxt, _L)]
        acc = jnp.zeros((_L,), jnp.float32)
        for j2 in range(_L):
            elem = o + j2
            slot = lax.rem(elem, _NBUF)
            drain(slot)
            w = wvec[j2]
            c = cvec[j2]
            slot16 = jnp.full((_L,), 0, jnp.int32) + slot
            colw = jnp.full((_L,), 0, jnp.int32) + (w & 127)
            colc = jnp.full((_L,), 0, jnp.int32) + (c & 127)
            p = jnp.zeros((_L,), jnp.float32)
            for q in range(_DIM // _L):
                rows = lane + q * _L
                wv = plsc.load_gather(wslabs, [slot16, rows, colw])
                cv = plsc.load_gather(cslabs, [slot16, rows, colc])
                p = p + wv * cv
            wb = plsc.load_gather(wbslabs, [slot16, zero16, colw])
            cb = plsc.load_gather(cbslabs, [slot16, zero16, colc])
            tot = jnp.sum(p) + wb[0] + cb[0]
            acc = jnp.where(lane == j2, tot, acc)
            if j2 < _L - _NBUF:
                wnext, cnext = wvec[j2 + _NBUF], cvec[j2 + _NBUF]
            else:
                wnext = jnp.where(last, wvec[_L - 1], wnxt[j2 - (_L - _NBUF)])
                cnext = jnp.where(last, cvec[_L - 1], cnxt[j2 - (_L - _NBUF)])
            fire(wnext, cnext, slot)
        out_v[pl.ds(o, _L)] = acc
        return carry

    lax.fori_loop(0, _NCHUNK, chunk_body, 0)

    for b in range(_NBUF):
        drain(b)

    pltpu.sync_copy(out_v, out_hbm.at[pl.ds(base, _BPW)])


@jax.jit
def kernel(word_ids, context_ids, word_embeddings, context_embeddings,
           word_biases, context_biases):
    mesh = plsc.VectorSubcoreMesh(core_axis_name="c", subcore_axis_name="s")
    run = functools.partial(
        pl.kernel,
        mesh=mesh,
        compiler_params=pltpu.CompilerParams(
            needs_layout_passes=False, disable_bounds_checks=True),
        out_type=jax.ShapeDtypeStruct((_BATCH,), jnp.float32),
        scratch_types=[
            pltpu.VMEM((_BPW,), jnp.int32),
            pltpu.VMEM((_BPW,), jnp.int32),
            pltpu.VMEM((_NBUF, _DIM, 128), jnp.float32),
            pltpu.VMEM((_NBUF, _DIM, 128), jnp.float32),
            pltpu.VMEM((_NBUF, 1, 128), jnp.float32),
            pltpu.VMEM((_NBUF, 1, 128), jnp.float32),
            pltpu.VMEM((_BPW,), jnp.float32),
            pltpu.SemaphoreType.DMA,
        ],
    )(_glove_kernel)
    return run(word_ids.astype(jnp.int32), context_ids.astype(jnp.int32),
               word_embeddings.T, context_embeddings.T,
               word_biases.T, context_biases.T)


# speedup vs baseline: 1.0408x; 1.0408x over previous
import functools

import jax
import jax.numpy as jnp
from jax import lax
from jax.experimental import pallas as pl
from jax.experimental.pallas import tpu as pltpu
from jax.experimental.pallas import tpu_sc as plsc

_VOCAB = 1000000
_DIM = 64
_BATCH = 16384

_INFO = plsc.get_sparse_core_info()
_NC = _INFO.num_cores
_NS = _INFO.num_subcores
_L = _INFO.num_lanes
_NW = _NC * _NS
_BPW = _BATCH // _NW
_NCHUNK = _BPW // _L
_NBUF = 4


def _glove_kernel(word_ids_hbm, ctx_ids_hbm, wembT_hbm, cembT_hbm,
                  wbT_hbm, cbT_hbm, out_hbm,
                  widx_v, cidx_v, wslabs, cslabs, wbslabs, cbslabs,
                  out_v, sem):
    wid = lax.axis_index("s") * _NC + lax.axis_index("c")
    base = wid * _BPW

    pltpu.sync_copy(word_ids_hbm.at[pl.ds(base, _BPW)], widx_v)
    pltpu.sync_copy(ctx_ids_hbm.at[pl.ds(base, _BPW)], cidx_v)

    def fire(w, c, slot):
        bw = pl.multiple_of((w >> 7) * 128, 128)
        bc = pl.multiple_of((c >> 7) * 128, 128)
        pltpu.async_copy(wembT_hbm.at[:, pl.ds(bw, 128)], wslabs[slot], sem)
        pltpu.async_copy(cembT_hbm.at[:, pl.ds(bc, 128)], cslabs[slot], sem)
        pltpu.async_copy(wbT_hbm.at[:, pl.ds(bw, 128)], wbslabs[slot], sem)
        pltpu.async_copy(cbT_hbm.at[:, pl.ds(bc, 128)], cbslabs[slot], sem)

    def drain(slot):
        pltpu.make_async_copy(
            wembT_hbm.at[:, pl.ds(0, 128)], wslabs[slot], sem).wait()
        pltpu.make_async_copy(
            cembT_hbm.at[:, pl.ds(0, 128)], cslabs[slot], sem).wait()
        pltpu.make_async_copy(
            wbT_hbm.at[:, pl.ds(0, 128)], wbslabs[slot], sem).wait()
        pltpu.make_async_copy(
            cbT_hbm.at[:, pl.ds(0, 128)], cbslabs[slot], sem).wait()

    lane = lax.iota(jnp.int32, _L)
    zero16 = jnp.zeros((_L,), jnp.int32)

    wvec0 = widx_v[pl.ds(0, _L)]
    cvec0 = cidx_v[pl.ds(0, _L)]
    for b in range(_NBUF):
        fire(wvec0[b], cvec0[b], b)

    def chunk_body(cc, carry):
        o = cc * _L
        wvec = widx_v[pl.ds(o, _L)]
        cvec = cidx_v[pl.ds(o, _L)]
        last = cc == _NCHUNK - 1
        onxt = jnp.where(last, o, o + _L)
        wnxt = widx_v[pl.ds(onxt, _L)]
        cnxt = cidx_v[pl.ds(onxt, _L)]
        acc = jnp.zeros((_L,), jnp.float32)
        for j2 in range(_L):
            slot = j2 % _NBUF
            drain(slot)
            w = wvec[j2]
            c = cvec[j2]
            colw = jnp.full((_L,), 0, jnp.int32) + (w & 127)
            colc = jnp.full((_L,), 0, jnp.int32) + (c & 127)
            p = jnp.zeros((_L,), jnp.float32)
            for q in range(_DIM // _L):
                rows = lane + q * _L
                wv = plsc.load_gather(wslabs[slot], [rows, colw])
                cv = plsc.load_gather(cslabs[slot], [rows, colc])
                p = p + wv * cv
            wb = plsc.load_gather(wbslabs[slot], [zero16, colw])
            cb = plsc.load_gather(cbslabs[slot], [zero16, colc])
            tot = jnp.sum(p) + wb[0] + cb[0]
            acc = jnp.where(lane == j2, tot, acc)
            if j2 < _L - _NBUF:
                wnext, cnext = wvec[j2 + _NBUF], cvec[j2 + _NBUF]
            else:
                wnext = jnp.where(last, wvec[_L - 1], wnxt[j2 - (_L - _NBUF)])
                cnext = jnp.where(last, cvec[_L - 1], cnxt[j2 - (_L - _NBUF)])
            fire(wnext, cnext, slot)
        out_v[pl.ds(o, _L)] = acc
        return carry

    lax.fori_loop(0, _NCHUNK, chunk_body, 0)

    for b in range(_NBUF):
        drain(b)

    pltpu.sync_copy(out_v, out_hbm.at[pl.ds(base, _BPW)])


@jax.jit
def kernel(word_ids, context_ids, word_embeddings, context_embeddings,
           word_biases, context_biases):
    mesh = plsc.VectorSubcoreMesh(core_axis_name="c", subcore_axis_name="s")

    def body(wi, ci, we, ce, wb, cb, out,
             widx_v, cidx_v,
             ws0, ws1, ws2, ws3, cs0, cs1, cs2, cs3,
             wb0, wb1, wb2, wb3, cb0, cb1, cb2, cb3, out_v, sem):
        _glove_kernel(wi, ci, we, ce, wb, cb, out,
                      widx_v, cidx_v,
                      [ws0, ws1, ws2, ws3], [cs0, cs1, cs2, cs3],
                      [wb0, wb1, wb2, wb3], [cb0, cb1, cb2, cb3],
                      out_v, sem)

    run = functools.partial(
        pl.kernel,
        mesh=mesh,
        compiler_params=pltpu.CompilerParams(
            needs_layout_passes=False, disable_bounds_checks=True),
        out_type=jax.ShapeDtypeStruct((_BATCH,), jnp.float32),
        scratch_types=[
            pltpu.VMEM((_BPW,), jnp.int32),
            pltpu.VMEM((_BPW,), jnp.int32),
        ] + [pltpu.VMEM((_DIM, 128), jnp.float32)] * (2 * _NBUF)
          + [pltpu.VMEM((1, 128), jnp.float32)] * (2 * _NBUF)
          + [
            pltpu.VMEM((_BPW,), jnp.float32),
            pltpu.SemaphoreType.DMA,
        ],
    )(body)
    return run(word_ids.astype(jnp.int32), context_ids.astype(jnp.int32),
               word_embeddings.T, context_embeddings.T,
               word_biases.T, context_biases.T)
